# Initial kernel scaffold; baseline (speedup 1.0000x reference)
import jax
import jax.numpy as jnp
from jax.experimental import pallas as pl


def kernel(x, edge_index, weights, W1l, b1l, W1r, b1r, att1, bias1, W2l, b2l, W2r, b2r, att2, bias2):
    raise NotImplementedError("write your pallas kernel here")


# speedup vs baseline: 22.5334x; 22.5334x over previous
import functools

import jax
import jax.numpy as jnp
from jax import lax
from jax.experimental import pallas as pl
from jax.experimental.pallas import tpu as pltpu
from jax.experimental.pallas import tpu_sc as plsc

N = 10000
D_IN = 128
H1, C1 = 2, 64
H2, C2 = 1, 4

NP = 10240
NSC, NTILE = 2, 16
NW = NSC * NTILE
B = 128
NT2 = NP // 16
RPT = NP // NTILE


def _proj1_body(x_ref, wl_ref, bl_ref, wr_ref, br_ref, xlr_ref):
    rows = lax.broadcasted_iota(jnp.int32, (NP, 1), 0)
    valid = rows < N
    x = x_ref[...]
    xl = jnp.dot(x, wl_ref[...], preferred_element_type=jnp.float32) + bl_ref[...]
    xr = jnp.dot(x, wr_ref[...], preferred_element_type=jnp.float32) + br_ref[...]
    xlr_ref[pl.ds(0, NP), :] = jnp.where(valid, xl, 0.0)
    xlr_ref[pl.ds(NP, NP), :] = jnp.where(valid, xr, 0.0)


def _mid_body(accn_ref, accd_ref, b1_ref, w2_ref, b2_ref, t2_ref):
    numer = accn_ref[0] + accn_ref[1]
    accd = accd_ref[0] + accd_ref[1]
    d0 = accd[:, 0:1]
    d1 = accd[:, C1:C1 + 1]
    d0 = jnp.where(d0 != 0.0, d0, 1.0)
    d1 = jnp.where(d1 != 0.0, d1, 1.0)
    div = jnp.concatenate([jnp.broadcast_to(d0, (NP, C1)),
                           jnp.broadcast_to(d1, (NP, C1))], axis=1)
    h = numer / div + b1_ref[...]
    h = jnp.maximum(h, 0.01 * h)
    t2t = lax.dot_general(w2_ref[...], h, (((0,), (1,)), ((), ())),
                          preferred_element_type=jnp.float32)
    t2t = t2t + b2_ref[...]
    cols = lax.broadcasted_iota(jnp.int32, (1, NP), 1)
    t2_ref[...] = jnp.where(cols < N, t2t, 0.0)


def _final_body(acc_ref, b2_ref, out_ref):
    a = acc_ref[0] + acc_ref[1]
    parts = []
    for sl in range(16):
        sub = a[:, 8 * sl:8 * sl + C2]
        den = a[:, 8 * sl + C2:8 * sl + C2 + 1]
        den = jnp.where(den != 0.0, den, 1.0)
        o = sub / den + b2_ref[...]
        m = jnp.max(o, axis=1, keepdims=True)
        e = jnp.exp(o - m)
        parts.append(e / jnp.sum(e, axis=1, keepdims=True))
    p = jnp.stack(parts, axis=1)
    out_ref[...] = p.reshape(NP, C2)[:N]


def _bc15(v):
    return v.at[jnp.full((16,), 15, jnp.int32)].get(mode="promise_in_bounds")


def _bc_lane(v, i):
    return v.at[jnp.full((16,), i, jnp.int32)].get(mode="promise_in_bounds")


def _make_edge1(ep, weighted):
    epw = ep // NW
    nchunk = epw // B
    mesh = plsc.VectorSubcoreMesh(core_axis_name="c", subcore_axis_name="s",
                                  num_cores=NSC, num_subcores=NTILE)

    @functools.partial(
        pl.kernel,
        out_type=jax.ShapeDtypeStruct((NSC, NP, 128), jnp.float32),
        mesh=mesh,
        scratch_types=[
            pltpu.VMEM((B,), jnp.int32),
            pltpu.VMEM((B,), jnp.int32),
            pltpu.VMEM((B,), jnp.int32),
            pltpu.VMEM((B, 128), jnp.float32),
            pltpu.VMEM((B, 128), jnp.float32),
            pltpu.VMEM((H1 * C1,), jnp.float32),
            pltpu.VMEM_SHARED((NP, 128), jnp.float32),
            pltpu.SemaphoreType.DMA,
            pltpu.SemaphoreType.DMA,
        ],
        compiler_params=pltpu.CompilerParams(needs_layout_passes=False),
    )
    def edge1(xlr_hbm, src_hbm, dst_hbm, att_hbm, acc_hbm,
              src_idx, dst_idx, dsg_idx, xl_rows, xr_rows, att_v, acc_sh,
              sem_a, sem_b):
        cid = lax.axis_index("c")
        sid = lax.axis_index("s")
        wid = cid * NTILE + sid

        def _zinit(g, cc):
            src_idx[pl.ds(g * 16, 16)] = jnp.full((16,), N, jnp.int32)
            return cc

        lax.fori_loop(0, B // 16, _zinit, 0)
        pltpu.async_copy(xlr_hbm.at[src_idx], xl_rows, sem_a).wait()
        for i in range(RPT // B):
            pltpu.sync_copy(xl_rows, acc_sh.at[pl.ds(sid * RPT + i * B, B)])
        plsc.subcore_barrier()

        def _chunk(t, c):
            base = wid * epw + t * B
            pltpu.sync_copy(src_hbm.at[pl.ds(base, B)], src_idx)
            pltpu.sync_copy(dst_hbm.at[pl.ds(base, B)], dst_idx)

            def _pidx(g, cc):
                dst_v = dst_idx[pl.ds(g * 16, 16)]
                dsg_idx[pl.ds(g * 16, 16)] = dst_v + NP
                return cc

            lax.fori_loop(0, B // 16, _pidx, 0)
            cpa = pltpu.async_copy(xlr_hbm.at[src_idx], xl_rows, sem_a)
            cpb = pltpu.async_copy(xlr_hbm.at[dsg_idx], xr_rows, sem_b)
            cpa.wait()
            cpb.wait()
            pltpu.sync_copy(att_hbm, att_v)

            def _edge(e, cc):
                xl = [xl_rows[e, pl.ds(16 * k, 16)] for k in range(8)]
                xr = [xr_rows[e, pl.ds(16 * k, 16)] for k in range(8)]
                attv = [att_v[pl.ds(16 * k, 16)] for k in range(8)]
                p = []
                for k in range(8):
                    m = xl[k] + xr[k]
                    ek = jnp.maximum(m, 0.2 * m)
                    p.append(ek * attv[k])
                h0 = (p[0] + p[1]) + (p[2] + p[3])
                h1 = (p[4] + p[5]) + (p[6] + p[7])
                w0 = jnp.exp(_bc15(plsc.cumsum(h0)))
                w1 = jnp.exp(_bc15(plsc.cumsum(h1)))
                if weighted:
                    for k in range(4):
                        xl_rows[e, pl.ds(16 * k, 16)] = w0 * xl[k]
                    for k in range(4, 8):
                        xl_rows[e, pl.ds(16 * k, 16)] = w1 * xl[k]
                else:
                    for k in range(4):
                        xl_rows[e, pl.ds(16 * k, 16)] = w0
                    for k in range(4, 8):
                        xl_rows[e, pl.ds(16 * k, 16)] = w1
                return cc

            lax.fori_loop(0, B, _edge, 0, unroll=2)
            pltpu.sync_copy(xl_rows, acc_sh.at[dst_idx], add=True)
            return c

        lax.fori_loop(0, nchunk, _chunk, 0)
        plsc.subcore_barrier()
        pltpu.sync_copy(acc_sh.at[pl.ds(sid * RPT, RPT)],
                        acc_hbm.at[cid, pl.ds(sid * RPT, RPT)])

    return edge1


def _make_edge2(ep):
    epw = ep // NW
    nchunk = epw // B
    mesh = plsc.VectorSubcoreMesh(core_axis_name="c", subcore_axis_name="s",
                                  num_cores=NSC, num_subcores=NTILE)

    @functools.partial(
        pl.kernel,
        out_type=jax.ShapeDtypeStruct((NSC, NT2, 128), jnp.float32),
        mesh=mesh,
        scratch_types=[
            pltpu.VMEM((B,), jnp.int32),
            pltpu.VMEM((B,), jnp.int32),
            pltpu.VMEM((B,), jnp.int32),
            pltpu.VMEM((8, NP), jnp.float32),
            pltpu.VMEM((B, 128), jnp.float32),
            pltpu.VMEM((16,), jnp.float32),
            pltpu.VMEM_SHARED((NT2, 128), jnp.float32),
        ],
        compiler_params=pltpu.CompilerParams(needs_layout_passes=False),
    )
    def edge2(t2_hbm, src_hbm, dst_hbm, att_hbm, acc_hbm,
              src_idx, dst_idx, acr_idx, t2_tile, out_rows, att_v, acc_sh):
        cid = lax.axis_index("c")
        sid = lax.axis_index("s")
        wid = cid * NTILE + sid
        zv = jnp.zeros((16,), jnp.float32)

        def _zrow(r, c):
            for k in range(8):
                out_rows[r, pl.ds(16 * k, 16)] = zv
            return c

        lax.fori_loop(0, B, _zrow, 0)
        rpt = NT2 // NTILE
        pltpu.sync_copy(out_rows.at[pl.ds(0, rpt)],
                        acc_sh.at[pl.ds(sid * rpt, rpt)])
        pltpu.sync_copy(t2_hbm, t2_tile)
        plsc.subcore_barrier()

        def _chunk(t, c):
            base = wid * epw + t * B
            pltpu.sync_copy(src_hbm.at[pl.ds(base, B)], src_idx)
            pltpu.sync_copy(dst_hbm.at[pl.ds(base, B)], dst_idx)
            pltpu.sync_copy(att_hbm, att_v)

            def _group(g, cc):
                att_all = att_v[pl.ds(0, 16)]
                attc = [_bc_lane(att_all, ci) for ci in range(C2)]
                rows16 = lax.iota(jnp.int32, 16) + g * 16
                src_v = src_idx[pl.ds(g * 16, 16)]
                dst_v = dst_idx[pl.ds(g * 16, 16)]
                drow = jnp.right_shift(dst_v, 4)
                dcol = jnp.bitwise_and(dst_v, 15) * 8
                acr_idx[pl.ds(g * 16, 16)] = drow
                xlc = []
                acc = jnp.zeros((16,), jnp.float32)
                for ci in range(C2):
                    a = plsc.load_gather(
                        t2_tile, [jnp.full((16,), ci, jnp.int32), src_v])
                    b = plsc.load_gather(
                        t2_tile, [jnp.full((16,), 4 + ci, jnp.int32), dst_v])
                    xlc.append(a)
                    m = a + b
                    ek = jnp.maximum(m, 0.2 * m)
                    acc = acc + attc[ci] * ek
                w = jnp.exp(acc)
                for ci in range(C2):
                    plsc.store_scatter(out_rows, [rows16, dcol + ci],
                                       w * xlc[ci])
                plsc.store_scatter(out_rows, [rows16, dcol + 4], w)
                return cc

            lax.fori_loop(0, B // 16, _group, 0)
            pltpu.sync_copy(out_rows, acc_sh.at[acr_idx], add=True)

            def _clr(g, cc):
                rows16 = lax.iota(jnp.int32, 16) + g * 16
                dst_v = dst_idx[pl.ds(g * 16, 16)]
                dcol = jnp.bitwise_and(dst_v, 15) * 8
                zz = jnp.zeros((16,), jnp.float32)
                for ci in range(C2 + 1):
                    plsc.store_scatter(out_rows, [rows16, dcol + ci], zz)
                return cc

            lax.fori_loop(0, B // 16, _clr, 0)
            return c

        lax.fori_loop(0, nchunk, _chunk, 0)
        plsc.subcore_barrier()
        rpt2 = NT2 // NTILE
        pltpu.sync_copy(acc_sh.at[pl.ds(sid * rpt2, rpt2)],
                        acc_hbm.at[cid, pl.ds(sid * rpt2, rpt2)])

    return edge2


def kernel(x, edge_index, weights, W1l, b1l, W1r, b1r, att1, bias1,
           W2l, b2l, W2r, b2r, att2, bias2):
    del weights
    n = x.shape[0]
    e_raw = edge_index.shape[1]
    et = e_raw + n
    ep = ((et + NW * B - 1) // (NW * B)) * (NW * B)
    loop = jnp.arange(n, dtype=edge_index.dtype)
    pad = jnp.full((ep - et,), n, jnp.int32)
    src = jnp.concatenate([edge_index[0], loop, pad])
    dst = jnp.concatenate([edge_index[1], loop, pad])
    xp = jnp.pad(x, ((0, NP - n), (0, 0)))
    xlr = pl.pallas_call(
        _proj1_body,
        out_shape=jax.ShapeDtypeStruct((2 * NP, H1 * C1), jnp.float32),
    )(xp, W1l, b1l, W1r, b1r)

    att1f = att1.reshape(H1 * C1)
    accn = _make_edge1(ep, True)(xlr, src, dst, att1f)
    accd = _make_edge1(ep, False)(xlr, src, dst, att1f)

    W2 = jnp.concatenate([W2l, W2r], axis=1)
    b2 = jnp.concatenate([b2l, b2r])
    t2p = pl.pallas_call(
        _mid_body,
        out_shape=jax.ShapeDtypeStruct((8, NP), jnp.float32),
    )(accn, accd, bias1, W2, b2.reshape(8, 1))

    att2f = jnp.pad(att2.reshape(H2 * C2), (0, 16 - H2 * C2))
    acc2 = _make_edge2(ep)(t2p, src, dst, att2f)

    out = pl.pallas_call(
        _final_body,
        out_shape=jax.ShapeDtypeStruct((N, C2), jnp.float32),
    )(acc2, bias2)
    return out


# speedup vs baseline: 23.9372x; 1.0623x over previous
import functools

import jax
import jax.numpy as jnp
from jax import lax
from jax.experimental import pallas as pl
from jax.experimental.pallas import tpu as pltpu
from jax.experimental.pallas import tpu_sc as plsc

N = 10000
D_IN = 128
H1, C1 = 2, 64
H2, C2 = 1, 4

NP = 10240
NSC, NTILE = 2, 16
NW = NSC * NTILE
B = 128
NT2 = NP // 16
RPT = NP // NTILE


def _proj1_body(x_ref, wl_ref, bl_ref, wr_ref, br_ref, xlr_ref):
    rows = lax.broadcasted_iota(jnp.int32, (NP, 1), 0)
    valid = rows < N
    x = x_ref[...]
    xl = jnp.dot(x, wl_ref[...], preferred_element_type=jnp.float32) + bl_ref[...]
    xr = jnp.dot(x, wr_ref[...], preferred_element_type=jnp.float32) + br_ref[...]
    xlr_ref[pl.ds(0, NP), :] = jnp.where(valid, xl, 0.0)
    xlr_ref[pl.ds(NP, NP), :] = jnp.where(valid, xr, 0.0)


def _mid_body(accn_ref, accd_ref, b1_ref, w2_ref, b2_ref, t2_ref):
    numer = accn_ref[0] + accn_ref[1]
    accd = accd_ref[0] + accd_ref[1]
    d0 = accd[:, 0:1]
    d1 = accd[:, C1:C1 + 1]
    d0 = jnp.where(d0 != 0.0, d0, 1.0)
    d1 = jnp.where(d1 != 0.0, d1, 1.0)
    div = jnp.concatenate([jnp.broadcast_to(d0, (NP, C1)),
                           jnp.broadcast_to(d1, (NP, C1))], axis=1)
    h = numer / div + b1_ref[...]
    h = jnp.maximum(h, 0.01 * h)
    t2t = lax.dot_general(w2_ref[...], h, (((0,), (1,)), ((), ())),
                          preferred_element_type=jnp.float32)
    t2t = t2t + b2_ref[...]
    cols = lax.broadcasted_iota(jnp.int32, (1, NP), 1)
    t2_ref[...] = jnp.where(cols < N, t2t, 0.0)


def _final_body(acc_ref, b2_ref, out_ref):
    a = acc_ref[0] + acc_ref[1]
    parts = []
    for sl in range(16):
        sub = a[:, 8 * sl:8 * sl + C2]
        den = a[:, 8 * sl + C2:8 * sl + C2 + 1]
        den = jnp.where(den != 0.0, den, 1.0)
        o = sub / den + b2_ref[...]
        m = jnp.max(o, axis=1, keepdims=True)
        e = jnp.exp(o - m)
        parts.append(e / jnp.sum(e, axis=1, keepdims=True))
    p = jnp.stack(parts, axis=1)
    out_ref[...] = p.reshape(NP, C2)[:N]


def _bc15(v):
    return v.at[jnp.full((16,), 15, jnp.int32)].get(mode="promise_in_bounds")


def _bc_lane(v, i):
    return v.at[jnp.full((16,), i, jnp.int32)].get(mode="promise_in_bounds")


def _make_edge1(ep, weighted):
    epw = ep // NW
    nchunk = epw // B
    mesh = plsc.VectorSubcoreMesh(core_axis_name="c", subcore_axis_name="s",
                                  num_cores=NSC, num_subcores=NTILE)

    @functools.partial(
        pl.kernel,
        out_type=jax.ShapeDtypeStruct((NSC, NP, 128), jnp.float32),
        mesh=mesh,
        scratch_types=[
            pltpu.VMEM((B,), jnp.int32),
            pltpu.VMEM((B,), jnp.int32),
            pltpu.VMEM((B,), jnp.int32),
            pltpu.VMEM((B, 128), jnp.float32),
            pltpu.VMEM((B, 128), jnp.float32),
            pltpu.VMEM((H1 * C1,), jnp.float32),
            pltpu.VMEM_SHARED((NP, 128), jnp.float32),
            pltpu.SemaphoreType.DMA,
            pltpu.SemaphoreType.DMA,
        ],
        compiler_params=pltpu.CompilerParams(needs_layout_passes=False),
    )
    def edge1(xlr_hbm, src_hbm, dst_hbm, att_hbm, acc_hbm,
              src_idx, dst_idx, dsg_idx, xl_rows, xr_rows, att_v, acc_sh,
              sem_a, sem_b):
        cid = lax.axis_index("c")
        sid = lax.axis_index("s")
        wid = cid * NTILE + sid

        def _zinit(g, cc):
            src_idx[pl.ds(g * 16, 16)] = jnp.full((16,), N, jnp.int32)
            return cc

        lax.fori_loop(0, B // 16, _zinit, 0)
        pltpu.sync_copy(att_hbm, att_v)
        pltpu.async_copy(xlr_hbm.at[src_idx], xl_rows, sem_a).wait()
        for i in range(RPT // B):
            pltpu.sync_copy(xl_rows, acc_sh.at[pl.ds(sid * RPT + i * B, B)])
        plsc.subcore_barrier()

        def _chunk(t, c):
            base = wid * epw + t * B
            pltpu.sync_copy(src_hbm.at[pl.ds(base, B)], src_idx)
            pltpu.sync_copy(dst_hbm.at[pl.ds(base, B)], dst_idx)

            def _pidx(g, cc):
                dst_v = dst_idx[pl.ds(g * 16, 16)]
                dsg_idx[pl.ds(g * 16, 16)] = dst_v + NP
                return cc

            lax.fori_loop(0, B // 16, _pidx, 0)
            cpa = pltpu.async_copy(xlr_hbm.at[src_idx], xl_rows, sem_a)
            cpb = pltpu.async_copy(xlr_hbm.at[dsg_idx], xr_rows, sem_b)
            cpa.wait()
            cpb.wait()

            def _edge(e, cc):
                xl = [xl_rows[e, pl.ds(16 * k, 16)] for k in range(8)]
                xr = [xr_rows[e, pl.ds(16 * k, 16)] for k in range(8)]
                attv = [att_v[pl.ds(16 * k, 16)] for k in range(8)]
                p = []
                for k in range(8):
                    m = xl[k] + xr[k]
                    ek = jnp.maximum(m, 0.2 * m)
                    p.append(ek * attv[k])
                h0 = (p[0] + p[1]) + (p[2] + p[3])
                h1 = (p[4] + p[5]) + (p[6] + p[7])
                w0 = jnp.exp(_bc15(plsc.cumsum(h0)))
                w1 = jnp.exp(_bc15(plsc.cumsum(h1)))
                if weighted:
                    for k in range(4):
                        xl_rows[e, pl.ds(16 * k, 16)] = w0 * xl[k]
                    for k in range(4, 8):
                        xl_rows[e, pl.ds(16 * k, 16)] = w1 * xl[k]
                else:
                    for k in range(4):
                        xl_rows[e, pl.ds(16 * k, 16)] = w0
                    for k in range(4, 8):
                        xl_rows[e, pl.ds(16 * k, 16)] = w1
                return cc

            lax.fori_loop(0, B, _edge, 0, unroll=2)
            pltpu.sync_copy(xl_rows, acc_sh.at[dst_idx], add=True)
            return c

        lax.fori_loop(0, nchunk, _chunk, 0)
        plsc.subcore_barrier()
        pltpu.sync_copy(acc_sh.at[pl.ds(sid * RPT, RPT)],
                        acc_hbm.at[cid, pl.ds(sid * RPT, RPT)])

    return edge1


def _make_edge2(ep):
    epw = ep // NW
    nchunk = epw // B
    mesh = plsc.VectorSubcoreMesh(core_axis_name="c", subcore_axis_name="s",
                                  num_cores=NSC, num_subcores=NTILE)

    @functools.partial(
        pl.kernel,
        out_type=jax.ShapeDtypeStruct((NSC, NT2, 128), jnp.float32),
        mesh=mesh,
        scratch_types=[
            pltpu.VMEM((B,), jnp.int32),
            pltpu.VMEM((B,), jnp.int32),
            pltpu.VMEM((B,), jnp.int32),
            pltpu.VMEM((8, NP), jnp.float32),
            pltpu.VMEM((B, 128), jnp.float32),
            pltpu.VMEM((16,), jnp.float32),
            pltpu.VMEM_SHARED((NT2, 128), jnp.float32),
        ],
        compiler_params=pltpu.CompilerParams(needs_layout_passes=False),
    )
    def edge2(t2_hbm, src_hbm, dst_hbm, att_hbm, acc_hbm,
              src_idx, dst_idx, acr_idx, t2_tile, out_rows, att_v, acc_sh):
        cid = lax.axis_index("c")
        sid = lax.axis_index("s")
        wid = cid * NTILE + sid
        zv = jnp.zeros((16,), jnp.float32)

        def _zrow(r, c):
            for k in range(8):
                out_rows[r, pl.ds(16 * k, 16)] = zv
            return c

        lax.fori_loop(0, B, _zrow, 0)
        rpt = NT2 // NTILE
        pltpu.sync_copy(out_rows.at[pl.ds(0, rpt)],
                        acc_sh.at[pl.ds(sid * rpt, rpt)])
        pltpu.sync_copy(t2_hbm, t2_tile)
        pltpu.sync_copy(att_hbm, att_v)
        plsc.subcore_barrier()

        def _chunk(t, c):
            base = wid * epw + t * B
            pltpu.sync_copy(src_hbm.at[pl.ds(base, B)], src_idx)
            pltpu.sync_copy(dst_hbm.at[pl.ds(base, B)], dst_idx)

            def _group(g, cc):
                att_all = att_v[pl.ds(0, 16)]
                attc = [_bc_lane(att_all, ci) for ci in range(C2)]
                rows16 = lax.iota(jnp.int32, 16) + g * 16
                src_v = src_idx[pl.ds(g * 16, 16)]
                dst_v = dst_idx[pl.ds(g * 16, 16)]
                drow = jnp.right_shift(dst_v, 4)
                dcol = jnp.bitwise_and(dst_v, 15) * 8
                acr_idx[pl.ds(g * 16, 16)] = drow
                xlc = []
                acc = jnp.zeros((16,), jnp.float32)
                for ci in range(C2):
                    a = plsc.load_gather(
                        t2_tile, [jnp.full((16,), ci, jnp.int32), src_v])
                    b = plsc.load_gather(
                        t2_tile, [jnp.full((16,), 4 + ci, jnp.int32), dst_v])
                    xlc.append(a)
                    m = a + b
                    ek = jnp.maximum(m, 0.2 * m)
                    acc = acc + attc[ci] * ek
                w = jnp.exp(acc)
                for ci in range(C2):
                    plsc.store_scatter(out_rows, [rows16, dcol + ci],
                                       w * xlc[ci])
                plsc.store_scatter(out_rows, [rows16, dcol + 4], w)
                return cc

            lax.fori_loop(0, B // 16, _group, 0)
            pltpu.sync_copy(out_rows, acc_sh.at[acr_idx], add=True)

            def _clr(g, cc):
                rows16 = lax.iota(jnp.int32, 16) + g * 16
                dst_v = dst_idx[pl.ds(g * 16, 16)]
                dcol = jnp.bitwise_and(dst_v, 15) * 8
                zz = jnp.zeros((16,), jnp.float32)
                for ci in range(C2 + 1):
                    plsc.store_scatter(out_rows, [rows16, dcol + ci], zz)
                return cc

            lax.fori_loop(0, B // 16, _clr, 0)
            return c

        lax.fori_loop(0, nchunk, _chunk, 0)
        plsc.subcore_barrier()
        rpt2 = NT2 // NTILE
        pltpu.sync_copy(acc_sh.at[pl.ds(sid * rpt2, rpt2)],
                        acc_hbm.at[cid, pl.ds(sid * rpt2, rpt2)])

    return edge2


def kernel(x, edge_index, weights, W1l, b1l, W1r, b1r, att1, bias1,
           W2l, b2l, W2r, b2r, att2, bias2):
    del weights
    n = x.shape[0]
    e_raw = edge_index.shape[1]
    et = e_raw + n
    ep = ((et + NW * B - 1) // (NW * B)) * (NW * B)
    loop = jnp.arange(n, dtype=edge_index.dtype)
    pad = jnp.full((ep - et,), n, jnp.int32)
    src = jnp.concatenate([edge_index[0], loop, pad])
    dst = jnp.concatenate([edge_index[1], loop, pad])
    xp = jnp.pad(x, ((0, NP - n), (0, 0)))
    xlr = pl.pallas_call(
        _proj1_body,
        out_shape=jax.ShapeDtypeStruct((2 * NP, H1 * C1), jnp.float32),
    )(xp, W1l, b1l, W1r, b1r)

    att1f = att1.reshape(H1 * C1)
    accn = _make_edge1(ep, True)(xlr, src, dst, att1f)
    accd = _make_edge1(ep, False)(xlr, src, dst, att1f)

    W2 = jnp.concatenate([W2l, W2r], axis=1)
    b2 = jnp.concatenate([b2l, b2r])
    t2p = pl.pallas_call(
        _mid_body,
        out_shape=jax.ShapeDtypeStruct((8, NP), jnp.float32),
    )(accn, accd, bias1, W2, b2.reshape(8, 1))

    att2f = jnp.pad(att2.reshape(H2 * C2), (0, 16 - H2 * C2))
    acc2 = _make_edge2(ep)(t2p, src, dst, att2f)

    out = pl.pallas_call(
        _final_body,
        out_shape=jax.ShapeDtypeStruct((N, C2), jnp.float32),
    )(acc2, bias2)
    return out


# speedup vs baseline: 24.7286x; 1.0331x over previous
import functools

import jax
import jax.numpy as jnp
from jax import lax
from jax.experimental import pallas as pl
from jax.experimental.pallas import tpu as pltpu
from jax.experimental.pallas import tpu_sc as plsc

N = 10000
D_IN = 128
H1, C1 = 2, 64
H2, C2 = 1, 4

NP = 10240
NSC, NTILE = 2, 16
NW = NSC * NTILE
B = 128
NT2 = NP // 16
RPT = NP // NTILE


def _proj1_body(x_ref, wl_ref, bl_ref, wr_ref, br_ref, xlr_ref):
    rows = lax.broadcasted_iota(jnp.int32, (NP, 1), 0)
    valid = rows < N
    x = x_ref[...]
    xl = jnp.dot(x, wl_ref[...], preferred_element_type=jnp.float32) + bl_ref[...]
    xr = jnp.dot(x, wr_ref[...], preferred_element_type=jnp.float32) + br_ref[...]
    xlr_ref[pl.ds(0, NP), :] = jnp.where(valid, xl, 0.0)
    xlr_ref[pl.ds(NP, NP), :] = jnp.where(valid, xr, 0.0)


def _mid_body(accn_ref, accd_ref, b1_ref, w2_ref, b2_ref, t2_ref):
    numer = accn_ref[0] + accn_ref[1]
    accd = accd_ref[0] + accd_ref[1]
    d0 = accd[:, 0:1]
    d1 = accd[:, C1:C1 + 1]
    d0 = jnp.where(d0 != 0.0, d0, 1.0)
    d1 = jnp.where(d1 != 0.0, d1, 1.0)
    div = jnp.concatenate([jnp.broadcast_to(d0, (NP, C1)),
                           jnp.broadcast_to(d1, (NP, C1))], axis=1)
    h = numer / div + b1_ref[...]
    h = jnp.maximum(h, 0.01 * h)
    t2t = lax.dot_general(w2_ref[...], h, (((0,), (1,)), ((), ())),
                          preferred_element_type=jnp.float32)
    t2t = t2t + b2_ref[...]
    cols = lax.broadcasted_iota(jnp.int32, (1, NP), 1)
    t2_ref[...] = jnp.where(cols < N, t2t, 0.0)


def _final_body(acc_ref, b2_ref, out_ref):
    a = acc_ref[0] + acc_ref[1]
    parts = []
    for sl in range(16):
        sub = a[:, 8 * sl:8 * sl + C2]
        den = a[:, 8 * sl + C2:8 * sl + C2 + 1]
        den = jnp.where(den != 0.0, den, 1.0)
        o = sub / den + b2_ref[...]
        m = jnp.max(o, axis=1, keepdims=True)
        e = jnp.exp(o - m)
        parts.append(e / jnp.sum(e, axis=1, keepdims=True))
    p = jnp.stack(parts, axis=1)
    out_ref[...] = p.reshape(NP, C2)[:N]


def _bc15(v):
    return v.at[jnp.full((16,), 15, jnp.int32)].get(mode="promise_in_bounds")


def _bc_lane(v, i):
    return v.at[jnp.full((16,), i, jnp.int32)].get(mode="promise_in_bounds")


def _make_edge1(ep, weighted):
    epw = ep // NW
    nchunk = epw // B
    mesh = plsc.VectorSubcoreMesh(core_axis_name="c", subcore_axis_name="s",
                                  num_cores=NSC, num_subcores=NTILE)

    @functools.partial(
        pl.kernel,
        out_type=jax.ShapeDtypeStruct((NSC, NP, 128), jnp.float32),
        mesh=mesh,
        scratch_types=[
            pltpu.VMEM((B,), jnp.int32),
            pltpu.VMEM((B,), jnp.int32),
            pltpu.VMEM((B,), jnp.int32),
            pltpu.VMEM((B, 128), jnp.float32),
            pltpu.VMEM((B, 128), jnp.float32),
            pltpu.VMEM((H1 * C1,), jnp.float32),
            pltpu.VMEM_SHARED((NP, 128), jnp.float32),
            pltpu.SemaphoreType.DMA,
            pltpu.SemaphoreType.DMA,
        ],
        compiler_params=pltpu.CompilerParams(needs_layout_passes=False),
    )
    def edge1(xlr_hbm, src_hbm, dst_hbm, att_hbm, acc_hbm,
              src_idx, dst_idx, dsg_idx, xl_rows, xr_rows, att_v, acc_sh,
              sem_a, sem_b):
        cid = lax.axis_index("c")
        sid = lax.axis_index("s")
        wid = cid * NTILE + sid

        def _zinit(g, cc):
            src_idx[pl.ds(g * 16, 16)] = jnp.full((16,), N, jnp.int32)
            return cc

        lax.fori_loop(0, B // 16, _zinit, 0)
        pltpu.sync_copy(att_hbm, att_v)
        pltpu.async_copy(xlr_hbm.at[src_idx], xl_rows, sem_a).wait()
        for i in range(RPT // B):
            pltpu.sync_copy(xl_rows, acc_sh.at[pl.ds(sid * RPT + i * B, B)])
        plsc.subcore_barrier()

        def _chunk(t, c):
            base = wid * epw + t * B
            ci1 = pltpu.async_copy(src_hbm.at[pl.ds(base, B)], src_idx, sem_a)
            ci2 = pltpu.async_copy(dst_hbm.at[pl.ds(base, B)], dst_idx, sem_b)
            ci1.wait()
            ci2.wait()

            def _pidx(g, cc):
                dst_v = dst_idx[pl.ds(g * 16, 16)]
                dsg_idx[pl.ds(g * 16, 16)] = dst_v + NP
                return cc

            lax.fori_loop(0, B // 16, _pidx, 0)
            cpa = pltpu.async_copy(xlr_hbm.at[src_idx], xl_rows, sem_a)
            cpb = pltpu.async_copy(xlr_hbm.at[dsg_idx], xr_rows, sem_b)
            cpa.wait()
            cpb.wait()

            def _edge(e, cc):
                xl = [xl_rows[e, pl.ds(16 * k, 16)] for k in range(8)]
                xr = [xr_rows[e, pl.ds(16 * k, 16)] for k in range(8)]
                attv = [att_v[pl.ds(16 * k, 16)] for k in range(8)]
                p = []
                for k in range(8):
                    m = xl[k] + xr[k]
                    ek = jnp.maximum(m, 0.2 * m)
                    p.append(ek * attv[k])
                h0 = (p[0] + p[1]) + (p[2] + p[3])
                h1 = (p[4] + p[5]) + (p[6] + p[7])
                w0 = jnp.exp(_bc15(plsc.cumsum(h0)))
                w1 = jnp.exp(_bc15(plsc.cumsum(h1)))
                if weighted:
                    for k in range(4):
                        xl_rows[e, pl.ds(16 * k, 16)] = w0 * xl[k]
                    for k in range(4, 8):
                        xl_rows[e, pl.ds(16 * k, 16)] = w1 * xl[k]
                else:
                    for k in range(4):
                        xl_rows[e, pl.ds(16 * k, 16)] = w0
                    for k in range(4, 8):
                        xl_rows[e, pl.ds(16 * k, 16)] = w1
                return cc

            lax.fori_loop(0, B, _edge, 0, unroll=2)
            pltpu.sync_copy(xl_rows, acc_sh.at[dst_idx], add=True)
            return c

        lax.fori_loop(0, nchunk, _chunk, 0)
        plsc.subcore_barrier()
        pltpu.sync_copy(acc_sh.at[pl.ds(sid * RPT, RPT)],
                        acc_hbm.at[cid, pl.ds(sid * RPT, RPT)])

    return edge1


def _make_edge2(ep):
    epw = ep // NW
    nchunk = epw // B
    mesh = plsc.VectorSubcoreMesh(core_axis_name="c", subcore_axis_name="s",
                                  num_cores=NSC, num_subcores=NTILE)

    @functools.partial(
        pl.kernel,
        out_type=jax.ShapeDtypeStruct((NSC, NT2, 128), jnp.float32),
        mesh=mesh,
        scratch_types=[
            pltpu.VMEM((B,), jnp.int32),
            pltpu.VMEM((B,), jnp.int32),
            pltpu.VMEM((B,), jnp.int32),
            pltpu.VMEM((8, NP), jnp.float32),
            pltpu.VMEM((B, 128), jnp.float32),
            pltpu.VMEM((16,), jnp.float32),
            pltpu.VMEM_SHARED((NT2, 128), jnp.float32),
        ],
        compiler_params=pltpu.CompilerParams(needs_layout_passes=False),
    )
    def edge2(t2_hbm, src_hbm, dst_hbm, att_hbm, acc_hbm,
              src_idx, dst_idx, acr_idx, t2_tile, out_rows, att_v, acc_sh):
        cid = lax.axis_index("c")
        sid = lax.axis_index("s")
        wid = cid * NTILE + sid
        zv = jnp.zeros((16,), jnp.float32)

        def _zrow(r, c):
            for k in range(8):
                out_rows[r, pl.ds(16 * k, 16)] = zv
            return c

        lax.fori_loop(0, B, _zrow, 0)
        rpt = NT2 // NTILE
        pltpu.sync_copy(out_rows.at[pl.ds(0, rpt)],
                        acc_sh.at[pl.ds(sid * rpt, rpt)])
        pltpu.sync_copy(t2_hbm, t2_tile)
        pltpu.sync_copy(att_hbm, att_v)
        plsc.subcore_barrier()

        def _chunk(t, c):
            base = wid * epw + t * B
            pltpu.sync_copy(src_hbm.at[pl.ds(base, B)], src_idx)
            pltpu.sync_copy(dst_hbm.at[pl.ds(base, B)], dst_idx)

            def _group(g, cc):
                att_all = att_v[pl.ds(0, 16)]
                attc = [_bc_lane(att_all, ci) for ci in range(C2)]
                rows16 = lax.iota(jnp.int32, 16) + g * 16
                src_v = src_idx[pl.ds(g * 16, 16)]
                dst_v = dst_idx[pl.ds(g * 16, 16)]
                drow = jnp.right_shift(dst_v, 4)
                dcol = jnp.bitwise_and(dst_v, 15) * 8
                acr_idx[pl.ds(g * 16, 16)] = drow
                xlc = []
                acc = jnp.zeros((16,), jnp.float32)
                for ci in range(C2):
                    a = plsc.load_gather(
                        t2_tile, [jnp.full((16,), ci, jnp.int32), src_v])
                    b = plsc.load_gather(
                        t2_tile, [jnp.full((16,), 4 + ci, jnp.int32), dst_v])
                    xlc.append(a)
                    m = a + b
                    ek = jnp.maximum(m, 0.2 * m)
                    acc = acc + attc[ci] * ek
                w = jnp.exp(acc)
                for ci in range(C2):
                    plsc.store_scatter(out_rows, [rows16, dcol + ci],
                                       w * xlc[ci])
                plsc.store_scatter(out_rows, [rows16, dcol + 4], w)
                return cc

            lax.fori_loop(0, B // 16, _group, 0)
            pltpu.sync_copy(out_rows, acc_sh.at[acr_idx], add=True)

            def _clr(g, cc):
                rows16 = lax.iota(jnp.int32, 16) + g * 16
                dst_v = dst_idx[pl.ds(g * 16, 16)]
                dcol = jnp.bitwise_and(dst_v, 15) * 8
                zz = jnp.zeros((16,), jnp.float32)
                for ci in range(C2 + 1):
                    plsc.store_scatter(out_rows, [rows16, dcol + ci], zz)
                return cc

            lax.fori_loop(0, B // 16, _clr, 0)
            return c

        lax.fori_loop(0, nchunk, _chunk, 0)
        plsc.subcore_barrier()
        rpt2 = NT2 // NTILE
        pltpu.sync_copy(acc_sh.at[pl.ds(sid * rpt2, rpt2)],
                        acc_hbm.at[cid, pl.ds(sid * rpt2, rpt2)])

    return edge2


def kernel(x, edge_index, weights, W1l, b1l, W1r, b1r, att1, bias1,
           W2l, b2l, W2r, b2r, att2, bias2):
    del weights
    n = x.shape[0]
    e_raw = edge_index.shape[1]
    et = e_raw + n
    ep = ((et + NW * B - 1) // (NW * B)) * (NW * B)
    loop = jnp.arange(n, dtype=edge_index.dtype)
    pad = jnp.full((ep - et,), n, jnp.int32)
    src = jnp.concatenate([edge_index[0], loop, pad])
    dst = jnp.concatenate([edge_index[1], loop, pad])
    xp = jnp.pad(x, ((0, NP - n), (0, 0)))
    xlr = pl.pallas_call(
        _proj1_body,
        out_shape=jax.ShapeDtypeStruct((2 * NP, H1 * C1), jnp.float32),
    )(xp, W1l, b1l, W1r, b1r)

    att1f = att1.reshape(H1 * C1)
    accn = _make_edge1(ep, True)(xlr, src, dst, att1f)
    accd = _make_edge1(ep, False)(xlr, src, dst, att1f)

    W2 = jnp.concatenate([W2l, W2r], axis=1)
    b2 = jnp.concatenate([b2l, b2r])
    t2p = pl.pallas_call(
        _mid_body,
        out_shape=jax.ShapeDtypeStruct((8, NP), jnp.float32),
    )(accn, accd, bias1, W2, b2.reshape(8, 1))

    att2f = jnp.pad(att2.reshape(H2 * C2), (0, 16 - H2 * C2))
    acc2 = _make_edge2(ep)(t2p, src, dst, att2f)

    out = pl.pallas_call(
        _final_body,
        out_shape=jax.ShapeDtypeStruct((N, C2), jnp.float32),
    )(acc2, bias2)
    return out
